# baseline (speedup 1.0000x reference)
import functools

import jax
import jax.numpy as jnp
from jax import lax
from jax.experimental import pallas as pl
from jax.experimental.pallas import tpu as pltpu
from jax.experimental.pallas import tpu_sc as plsc

N_NODES = 10000
DILATION = 2
T_IN = 12
T_OUT = 10
C = 32
DC = 32
E_EDGES = 320000

NC, NS, L = 2, 16, 16
NPAD = 10240
BLK = 512
NBLK = NPAD // BLK
HW = 128
EPT = 20480
EPAD = EPT * NS
K = 64
NCHUNK = EPT // K
SUBC = 32
NSUB = NCHUNK // SUBC
GRP = SUBC // 4
Q = 4
QW = (T_OUT * DC) // Q
RPT = NPAD // NS
EPT1 = EPAD // (NC * NS)


def _pieces(t):
    g0 = DC * t
    g = g0
    out = []
    while g < g0 + DC:
        q, off = divmod(g, QW)
        w = min(QW - off, g0 + DC - g)
        out.append((q, off, g - g0, w))
        g += w
    return out


EB = 65536

def _prep_kernel(s_ref, d_ref, so_ref, do_ref):
    i = pl.program_id(0)
    ids = i * EB + lax.broadcasted_iota(jnp.int32, (EB // 128, 128), 0) * 128 \
        + lax.broadcasted_iota(jnp.int32, (EB // 128, 128), 1)
    m = ids < E_EDGES
    so_ref[...] = jnp.where(m, s_ref[0, 0, :].reshape(EB // 128, 128), N_NODES)
    do_ref[...] = jnp.where(m, d_ref[0, 0, :].reshape(EB // 128, 128), N_NODES)


def _tc_edge_prep(e3):
    out = pl.pallas_call(
        _prep_kernel,
        grid=(EPAD // EB,),
        in_specs=[pl.BlockSpec((1, 1, EB), lambda i: (0, 0, i)),
                  pl.BlockSpec((1, 1, EB), lambda i: (1, 0, i))],
        out_specs=[pl.BlockSpec((EB // 128, 128), lambda i: (i, 0)),
                   pl.BlockSpec((EB // 128, 128), lambda i: (i, 0))],
        out_shape=[jax.ShapeDtypeStruct((EPAD // 128, 128), jnp.int32),
                   jax.ShapeDtypeStruct((EPAD // 128, 128), jnp.int32)],
    )(e3, e3)
    return out


def _deg_kernel(dst_hbm, out_hbm, dstv, acc):
    c = lax.axis_index("c")
    s = lax.axis_index("s")
    wid = s * NC + c

    pltpu.sync_copy(dst_hbm.at[wid], dstv)

    def zero(i, carry):
        acc[pl.ds(i * L, L)] = jnp.zeros((L,), jnp.float32)
        return carry

    lax.fori_loop(0, NPAD // L, zero, 0)

    ones = jnp.full((L,), 1.0, jnp.float32)

    def body(i, carry):
        idx = dstv[pl.ds(i * L, L)]
        plsc.addupdate_scatter(acc, [idx], ones)
        return carry

    lax.fori_loop(0, EPT1 // L, body, 0)
    pltpu.sync_copy(acc, out_hbm.at[wid])


def _sc_degree(dst2d):
    mesh = plsc.VectorSubcoreMesh(core_axis_name="c", subcore_axis_name="s")
    return pl.kernel(
        _deg_kernel,
        out_type=jax.ShapeDtypeStruct((NC * NS, NPAD), jnp.float32),
        mesh=mesh,
        compiler_params=pltpu.CompilerParams(needs_layout_passes=False),
        scratch_types=[
            pltpu.VMEM((EPT1,), jnp.int32),
            pltpu.VMEM((NPAD,), jnp.float32),
        ],
    )(dst2d)


def _dense_kernel(x_ref, degp_ref, wa, wb, wg, bias, h2_ref, dinv_ref):
    i = pl.program_id(0)
    deg = jnp.sum(degp_ref[...], axis=0) + 1.0
    dinv = lax.rsqrt(deg)
    dinv_ref[0, 0, :] = dinv

    rowids = i * BLK + lax.broadcasted_iota(jnp.int32, (BLK, 1), 0)
    valid = (rowids < N_NODES)[None, :, :]

    x = x_ref[...]
    xa = x[0:T_OUT].reshape(T_OUT * BLK, C)
    xb = x[DILATION:T_IN].reshape(T_OUT * BLK, C)
    p = jnp.dot(xa, wa[...], preferred_element_type=jnp.float32)
    p += jnp.dot(xb, wb[...], preferred_element_type=jnp.float32)
    g = jnp.tanh(p[:, :DC] + bias[0:1, :]) * jax.nn.sigmoid(p[:, DC:] + bias[1:2, :])
    h = jnp.dot(g, wg[...], preferred_element_type=jnp.float32)
    h = h.reshape(T_OUT, BLK, C) * dinv[None, :, None]
    h = jnp.where(valid, h, 0.0)
    for t in range(T_OUT):
        for (q, off, ho, w) in _pieces(t):
            h2_ref[q, :, off:off + w] = h[t, :, ho:ho + w]


def _tc_dense(x, degp, wa, wb, wg, bias):
    full = lambda shape: pl.BlockSpec(shape, lambda i: tuple(0 for _ in shape))
    return pl.pallas_call(
        _dense_kernel,
        grid=(NBLK,),
        in_specs=[
            pl.BlockSpec((T_IN, BLK, C), lambda i: (0, i, 0)),
            pl.BlockSpec((NC * NS, BLK), lambda i: (0, i)),
            full((C, 2 * DC)), full((C, 2 * DC)),
            full((DC, C)), full((2, DC)),
        ],
        out_specs=[
            pl.BlockSpec((Q, BLK, HW), lambda i: (0, i, 0)),
            pl.BlockSpec((1, 1, BLK), lambda i: (i, 0, 0)),
        ],
        out_shape=[
            jax.ShapeDtypeStruct((Q, NPAD, HW), jnp.float32),
            jax.ShapeDtypeStruct((NBLK, 1, BLK), jnp.float32),
        ],
    )(x, degp, wa, wb, wg, bias)


def _agg_kernel(h2_hbm, src_hbm, dst_hbm, agg_hbm, idx_src, idx_dst,
                rows_a, rows_b, rows_h, rows_hs, sem_a, sem_b, sem_h, table, acc):
    c = lax.axis_index("c")
    s = lax.axis_index("s")
    srcs = src_hbm.at[s]
    dsts = dst_hbm.at[s]
    rbase = s * RPT

    for p in range(2):
        q = 2 * c + p
        h2q = h2_hbm.at[q]
        pltpu.sync_copy(h2q.at[pl.ds(rbase, RPT), pl.ds(0, QW)],
                        table.at[pl.ds(rbase, RPT)])
        pltpu.sync_copy(h2q.at[pl.ds(rbase, RPT), pl.ds(0, QW)],
                        acc.at[pl.ds(rbase, RPT)])
        plsc.subcore_barrier()

        h2wide = h2_hbm.at[q]

        def outer(b, carry):
            pltpu.sync_copy(srcs.at[pl.ds(b * SUBC, SUBC)], idx_src)
            pltpu.sync_copy(dsts.at[pl.ds(b * SUBC, SUBC)], idx_dst)
            pltpu.async_copy(table.at[idx_src.at[0]], rows_a, sem_a)
            pltpu.async_copy(h2wide.at[idx_src.at[3]], rows_h, sem_h)

            def body(g, carry2):
                j = 4 * g
                pltpu.async_copy(table.at[idx_src.at[j + 1]], rows_b, sem_b)
                pltpu.make_async_copy(table.at[idx_src.at[j]], rows_a, sem_a).wait()
                pltpu.sync_copy(rows_a, acc.at[idx_dst.at[j]], add=True)
                pltpu.async_copy(table.at[idx_src.at[j + 2]], rows_a, sem_a)
                pltpu.make_async_copy(table.at[idx_src.at[j + 1]], rows_b, sem_b).wait()
                pltpu.sync_copy(rows_b, acc.at[idx_dst.at[j + 1]], add=True)
                pltpu.make_async_copy(table.at[idx_src.at[j + 2]], rows_a, sem_a).wait()
                pltpu.sync_copy(rows_a, acc.at[idx_dst.at[j + 2]], add=True)
                pltpu.make_async_copy(h2wide.at[idx_src.at[j + 3]], rows_h, sem_h).wait()

                def repack(r, carry3):
                    for cc in range(QW // L):
                        rows_hs[r, pl.ds(cc * L, L)] = rows_h[r, pl.ds(cc * L, L)]
                    return carry3

                lax.fori_loop(0, K, repack, 0)
                pltpu.sync_copy(rows_hs, acc.at[idx_dst.at[j + 3]], add=True)

                @pl.when(g + 1 < GRP)
                def _():
                    pltpu.async_copy(table.at[idx_src.at[j + 4]], rows_a, sem_a)
                    pltpu.async_copy(h2wide.at[idx_src.at[j + 7]], rows_h, sem_h)

                return carry2

            lax.fori_loop(0, GRP, body, carry)
            return carry

        lax.fori_loop(0, NSUB, outer, 0)
        plsc.subcore_barrier()
        pltpu.sync_copy(acc.at[pl.ds(rbase, RPT)],
                        agg_hbm.at[q].at[pl.ds(rbase, RPT), pl.ds(0, QW)])


def _sc_aggregate(h2, src3d, dst3d):
    mesh = plsc.VectorSubcoreMesh(core_axis_name="c", subcore_axis_name="s")
    return pl.kernel(
        _agg_kernel,
        out_type=jax.ShapeDtypeStruct((Q, NPAD, HW), jnp.float32),
        mesh=mesh,
        compiler_params=pltpu.CompilerParams(use_tc_tiling_on_sc=False),
        scratch_types=[
            pltpu.VMEM((SUBC, K), jnp.int32),
            pltpu.VMEM((SUBC, K), jnp.int32),
            pltpu.VMEM((K, QW), jnp.float32),
            pltpu.VMEM((K, QW), jnp.float32),
            pltpu.VMEM((K, HW), jnp.float32),
            pltpu.VMEM((K, QW), jnp.float32),
            pltpu.SemaphoreType.DMA,
            pltpu.SemaphoreType.DMA,
            pltpu.SemaphoreType.DMA,
            pltpu.VMEM_SHARED((NPAD, QW), jnp.float32),
            pltpu.VMEM_SHARED((NPAD, QW), jnp.float32),
        ],
    )(h2, src3d, dst3d)


def _final_kernel(agg_ref, dinv_ref, w2, bias, out_ref):
    dinv_col = dinv_ref[0, 0, :][:, None]
    gb = bias[0:1, :]
    ob = bias[1:2, :]
    for t in range(T_OUT):
        pieces = [agg_ref[q, :, off:off + w] for (q, off, ho, w) in _pieces(t)]
        a = pieces[0] if len(pieces) == 1 else jnp.concatenate(pieces, axis=1)
        a = a * dinv_col + gb
        out_ref[t, :, :] = jnp.dot(a, w2[...], preferred_element_type=jnp.float32) + ob


def _tc_final(agg, dinv2d, w2, bias):
    full = lambda shape: pl.BlockSpec(shape, lambda i: tuple(0 for _ in shape))
    return pl.pallas_call(
        _final_kernel,
        grid=(NBLK,),
        in_specs=[
            pl.BlockSpec((Q, BLK, HW), lambda i: (0, i, 0)),
            pl.BlockSpec((1, 1, BLK), lambda i: (i, 0, 0)),
            full((C, C)), full((2, C)),
        ],
        out_specs=pl.BlockSpec((T_OUT, BLK, C), lambda i: (0, i, 0)),
        out_shape=jax.ShapeDtypeStruct((T_OUT, N_NODES, C), jnp.float32),
    )(agg, dinv2d, w2, bias)


def kernel(input, edge_index, gate1_w, gate1_b, gate2_w, gate2_b, gcn_w, gcn_b, out_w, out_b):
    x = input[0]
    src, dst = _tc_edge_prep(edge_index.reshape(2, 1, E_EDGES))
    dst2d = dst.reshape(NC * NS, EPT1)
    src3d = src.reshape(NS, NCHUNK, K)
    dst3d = dst.reshape(NS, NCHUNK, K)

    wa = jnp.concatenate([gate1_w[:, :, 0, 0].T, gate2_w[:, :, 0, 0].T], axis=1)
    wb = jnp.concatenate([gate1_w[:, :, 0, 1].T, gate2_w[:, :, 0, 1].T], axis=1)
    wg = gcn_w.T
    gbias = jnp.stack([gate1_b, gate2_b])
    w2 = out_w[:, :, 0, 0].T
    fbias = jnp.stack([gcn_b, out_b])

    degp = _sc_degree(dst2d)
    h2, dinv2d = _tc_dense(x, degp, wa, wb, wg, gbias)
    agg = _sc_aggregate(h2, src3d, dst3d)
    out = _tc_final(agg, dinv2d, w2, fbias)
    return out[None]


# speedup vs baseline: 1.4704x; 1.4704x over previous
import functools

import jax
import jax.numpy as jnp
from jax import lax
from jax.experimental import pallas as pl
from jax.experimental.pallas import tpu as pltpu
from jax.experimental.pallas import tpu_sc as plsc

N_NODES = 10000
DILATION = 2
T_IN = 12
T_OUT = 10
C = 32
DC = 32
E_EDGES = 320000

NC, NS, L = 2, 16, 16
NPAD = 10240
BLK = 512
NBLK = NPAD // BLK
HW = 128
EPT = 20480
EPAD = EPT * NS
K = 128
NCHUNK = EPT // K
SUBC = 16
NSUB = NCHUNK // SUBC
Q = 4
QW = (T_OUT * DC) // Q
RPT = NPAD // NS
EPT1 = EPAD // (NC * NS)


def _pieces(t):
    g0 = DC * t
    g = g0
    out = []
    while g < g0 + DC:
        q, off = divmod(g, QW)
        w = min(QW - off, g0 + DC - g)
        out.append((q, off, g - g0, w))
        g += w
    return out


EB = 65536

def _prep_kernel(s_ref, d_ref, so_ref, do_ref):
    i = pl.program_id(0)
    ids = i * EB + lax.broadcasted_iota(jnp.int32, (EB // 128, 128), 0) * 128 \
        + lax.broadcasted_iota(jnp.int32, (EB // 128, 128), 1)
    m = ids < E_EDGES
    so_ref[...] = jnp.where(m, s_ref[0, 0, :].reshape(EB // 128, 128), N_NODES)
    do_ref[...] = jnp.where(m, d_ref[0, 0, :].reshape(EB // 128, 128), N_NODES)


def _tc_edge_prep(e3):
    out = pl.pallas_call(
        _prep_kernel,
        grid=(EPAD // EB,),
        in_specs=[pl.BlockSpec((1, 1, EB), lambda i: (0, 0, i)),
                  pl.BlockSpec((1, 1, EB), lambda i: (1, 0, i))],
        out_specs=[pl.BlockSpec((EB // 128, 128), lambda i: (i, 0)),
                   pl.BlockSpec((EB // 128, 128), lambda i: (i, 0))],
        out_shape=[jax.ShapeDtypeStruct((EPAD // 128, 128), jnp.int32),
                   jax.ShapeDtypeStruct((EPAD // 128, 128), jnp.int32)],
    )(e3, e3)
    return out


def _deg_kernel(dst_hbm, out_hbm, dstv, acc):
    c = lax.axis_index("c")
    s = lax.axis_index("s")
    wid = s * NC + c

    pltpu.sync_copy(dst_hbm.at[wid], dstv)

    def zero(i, carry):
        acc[pl.ds(i * L, L)] = jnp.zeros((L,), jnp.float32)
        return carry

    lax.fori_loop(0, NPAD // L, zero, 0)

    ones = jnp.full((L,), 1.0, jnp.float32)

    def body(i, carry):
        idx = dstv[pl.ds(i * L, L)]
        plsc.addupdate_scatter(acc, [idx], ones)
        return carry

    lax.fori_loop(0, EPT1 // L, body, 0)
    pltpu.sync_copy(acc, out_hbm.at[wid])


def _sc_degree(dst2d):
    mesh = plsc.VectorSubcoreMesh(core_axis_name="c", subcore_axis_name="s")
    return pl.kernel(
        _deg_kernel,
        out_type=jax.ShapeDtypeStruct((NC * NS, NPAD), jnp.float32),
        mesh=mesh,
        compiler_params=pltpu.CompilerParams(needs_layout_passes=False),
        scratch_types=[
            pltpu.VMEM((EPT1,), jnp.int32),
            pltpu.VMEM((NPAD,), jnp.float32),
        ],
    )(dst2d)


def _dense_kernel(x_ref, degp_ref, wa, wb, wg, bias, h2_ref, dinv_ref):
    i = pl.program_id(0)
    deg = jnp.sum(degp_ref[...], axis=0) + 1.0
    dinv = lax.rsqrt(deg)
    dinv_ref[0, 0, :] = dinv

    rowids = i * BLK + lax.broadcasted_iota(jnp.int32, (BLK, 1), 0)
    valid = (rowids < N_NODES)[None, :, :]

    x = x_ref[...]
    xa = x[0:T_OUT].reshape(T_OUT * BLK, C)
    xb = x[DILATION:T_IN].reshape(T_OUT * BLK, C)
    p = jnp.dot(xa, wa[...], preferred_element_type=jnp.float32)
    p += jnp.dot(xb, wb[...], preferred_element_type=jnp.float32)
    g = jnp.tanh(p[:, :DC] + bias[0:1, :]) * jax.nn.sigmoid(p[:, DC:] + bias[1:2, :])
    h = jnp.dot(g, wg[...], preferred_element_type=jnp.float32)
    h = h.reshape(T_OUT, BLK, C) * dinv[None, :, None]
    h = jnp.where(valid, h, 0.0)
    for t in range(T_OUT):
        for (q, off, ho, w) in _pieces(t):
            h2_ref[q, :, off:off + w] = h[t, :, ho:ho + w]


def _tc_dense(x, degp, wa, wb, wg, bias):
    full = lambda shape: pl.BlockSpec(shape, lambda i: tuple(0 for _ in shape))
    return pl.pallas_call(
        _dense_kernel,
        grid=(NBLK,),
        in_specs=[
            pl.BlockSpec((T_IN, BLK, C), lambda i: (0, i, 0)),
            pl.BlockSpec((NC * NS, BLK), lambda i: (0, i)),
            full((C, 2 * DC)), full((C, 2 * DC)),
            full((DC, C)), full((2, DC)),
        ],
        out_specs=[
            pl.BlockSpec((Q, BLK, HW), lambda i: (0, i, 0)),
            pl.BlockSpec((1, 1, BLK), lambda i: (i, 0, 0)),
        ],
        out_shape=[
            jax.ShapeDtypeStruct((Q, NPAD, HW), jnp.float32),
            jax.ShapeDtypeStruct((NBLK, 1, BLK), jnp.float32),
        ],
    )(x, degp, wa, wb, wg, bias)


def _agg_kernel(h2_hbm, src_hbm, dst_hbm, agg_hbm, idx_src, idx_dst,
                rows_a, rows_b, sem_a, sem_b, table, acc):
    c = lax.axis_index("c")
    s = lax.axis_index("s")
    srcs = src_hbm.at[s]
    dsts = dst_hbm.at[s]
    rbase = s * RPT

    for p in range(2):
        q = 2 * c + p
        h2q = h2_hbm.at[q]
        pltpu.sync_copy(h2q.at[pl.ds(rbase, RPT), pl.ds(0, QW)],
                        table.at[pl.ds(rbase, RPT)])
        pltpu.sync_copy(h2q.at[pl.ds(rbase, RPT), pl.ds(0, QW)],
                        acc.at[pl.ds(rbase, RPT)])
        plsc.subcore_barrier()

        def outer(b, carry):
            pltpu.sync_copy(srcs.at[pl.ds(b * SUBC, SUBC)], idx_src)
            pltpu.sync_copy(dsts.at[pl.ds(b * SUBC, SUBC)], idx_dst)
            pltpu.async_copy(table.at[idx_src.at[0]], rows_a, sem_a)

            def body(jj, carry2):
                j0 = 2 * jj
                pltpu.async_copy(table.at[idx_src.at[j0 + 1]], rows_b, sem_b)
                pltpu.make_async_copy(table.at[idx_src.at[j0]], rows_a, sem_a).wait()
                pltpu.sync_copy(rows_a, acc.at[idx_dst.at[j0]], add=True)

                @pl.when(j0 + 2 < SUBC)
                def _():
                    pltpu.async_copy(table.at[idx_src.at[j0 + 2]], rows_a, sem_a)

                pltpu.make_async_copy(table.at[idx_src.at[j0 + 1]], rows_b, sem_b).wait()
                pltpu.sync_copy(rows_b, acc.at[idx_dst.at[j0 + 1]], add=True)
                return carry2

            lax.fori_loop(0, SUBC // 2, body, carry)
            return carry

        lax.fori_loop(0, NSUB, outer, 0)
        plsc.subcore_barrier()
        pltpu.sync_copy(acc.at[pl.ds(rbase, RPT)],
                        agg_hbm.at[q].at[pl.ds(rbase, RPT), pl.ds(0, QW)])


def _sc_aggregate(h2, src3d, dst3d):
    mesh = plsc.VectorSubcoreMesh(core_axis_name="c", subcore_axis_name="s")
    return pl.kernel(
        _agg_kernel,
        out_type=jax.ShapeDtypeStruct((Q, NPAD, HW), jnp.float32),
        mesh=mesh,
        compiler_params=pltpu.CompilerParams(use_tc_tiling_on_sc=False),
        scratch_types=[
            pltpu.VMEM((SUBC, K), jnp.int32),
            pltpu.VMEM((SUBC, K), jnp.int32),
            pltpu.VMEM((K, QW), jnp.float32),
            pltpu.VMEM((K, QW), jnp.float32),
            pltpu.SemaphoreType.DMA,
            pltpu.SemaphoreType.DMA,
            pltpu.VMEM_SHARED((NPAD, QW), jnp.float32),
            pltpu.VMEM_SHARED((NPAD, QW), jnp.float32),
        ],
    )(h2, src3d, dst3d)


def _final_kernel(agg_ref, dinv_ref, w2, bias, out_ref):
    dinv_col = dinv_ref[0, 0, :][:, None]
    gb = bias[0:1, :]
    ob = bias[1:2, :]
    for t in range(T_OUT):
        pieces = [agg_ref[q, :, off:off + w] for (q, off, ho, w) in _pieces(t)]
        a = pieces[0] if len(pieces) == 1 else jnp.concatenate(pieces, axis=1)
        a = a * dinv_col + gb
        out_ref[t, :, :] = jnp.dot(a, w2[...], preferred_element_type=jnp.float32) + ob


def _tc_final(agg, dinv2d, w2, bias):
    full = lambda shape: pl.BlockSpec(shape, lambda i: tuple(0 for _ in shape))
    return pl.pallas_call(
        _final_kernel,
        grid=(NBLK,),
        in_specs=[
            pl.BlockSpec((Q, BLK, HW), lambda i: (0, i, 0)),
            pl.BlockSpec((1, 1, BLK), lambda i: (i, 0, 0)),
            full((C, C)), full((2, C)),
        ],
        out_specs=pl.BlockSpec((T_OUT, BLK, C), lambda i: (0, i, 0)),
        out_shape=jax.ShapeDtypeStruct((T_OUT, N_NODES, C), jnp.float32),
    )(agg, dinv2d, w2, bias)


def kernel(input, edge_index, gate1_w, gate1_b, gate2_w, gate2_b, gcn_w, gcn_b, out_w, out_b):
    x = input[0]
    src, dst = _tc_edge_prep(edge_index.reshape(2, 1, E_EDGES))
    dst2d = dst.reshape(NC * NS, EPT1)
    src3d = src.reshape(NS, NCHUNK, K)
    dst3d = dst.reshape(NS, NCHUNK, K)

    wa = jnp.concatenate([gate1_w[:, :, 0, 0].T, gate2_w[:, :, 0, 0].T], axis=1)
    wb = jnp.concatenate([gate1_w[:, :, 0, 1].T, gate2_w[:, :, 0, 1].T], axis=1)
    wg = gcn_w.T
    gbias = jnp.stack([gate1_b, gate2_b])
    w2 = out_w[:, :, 0, 0].T
    fbias = jnp.stack([gcn_b, out_b])

    degp = _sc_degree(dst2d)
    h2, dinv2d = _tc_dense(x, degp, wa, wb, wg, gbias)
    agg = _sc_aggregate(h2, src3d, dst3d)
    out = _tc_final(agg, dinv2d, w2, fbias)
    return out[None]


# speedup vs baseline: 1.4887x; 1.0125x over previous
import functools

import jax
import jax.numpy as jnp
from jax import lax
from jax.experimental import pallas as pl
from jax.experimental.pallas import tpu as pltpu
from jax.experimental.pallas import tpu_sc as plsc

N_NODES = 10000
DILATION = 2
T_IN = 12
T_OUT = 10
C = 32
DC = 32
E_EDGES = 320000

NC, NS, L = 2, 16, 16
NPAD = 10240
BLK = 512
NBLK = NPAD // BLK
HW = 128
EPT = 20480
EPAD = EPT * NS
K = 128
NCHUNK = EPT // K
SUBC = 16
NSUB = NCHUNK // SUBC
Q = 4
QW = (T_OUT * DC) // Q
RPT = NPAD // NS
EPT1 = E_EDGES // (NC * NS)


def _pieces(t):
    g0 = DC * t
    g = g0
    out = []
    while g < g0 + DC:
        q, off = divmod(g, QW)
        w = min(QW - off, g0 + DC - g)
        out.append((q, off, g - g0, w))
        g += w
    return out


EB = 65536

def _prep_kernel(s_ref, d_ref, so_ref, do_ref):
    i = pl.program_id(0)
    ids = i * EB + lax.broadcasted_iota(jnp.int32, (EB // 128, 128), 0) * 128 \
        + lax.broadcasted_iota(jnp.int32, (EB // 128, 128), 1)
    m = ids < E_EDGES
    so_ref[...] = jnp.where(m, s_ref[0, 0, :].reshape(EB // 128, 128), N_NODES)
    do_ref[...] = jnp.where(m, d_ref[0, 0, :].reshape(EB // 128, 128), N_NODES)


def _tc_edge_prep(e3):
    out = pl.pallas_call(
        _prep_kernel,
        grid=(EPAD // EB,),
        in_specs=[pl.BlockSpec((1, 1, EB), lambda i: (0, 0, i)),
                  pl.BlockSpec((1, 1, EB), lambda i: (1, 0, i))],
        out_specs=[pl.BlockSpec((EB // 128, 128), lambda i: (i, 0)),
                   pl.BlockSpec((EB // 128, 128), lambda i: (i, 0))],
        out_shape=[jax.ShapeDtypeStruct((EPAD // 128, 128), jnp.int32),
                   jax.ShapeDtypeStruct((EPAD // 128, 128), jnp.int32)],
    )(e3, e3)
    return out


def _deg_kernel(e_hbm, out_hbm, dstv, acc):
    c = lax.axis_index("c")
    s = lax.axis_index("s")
    wid = s * NC + c

    pltpu.sync_copy(e_hbm.at[1, 0].at[pl.ds(wid * EPT1, EPT1)], dstv)

    def zero(i, carry):
        acc[pl.ds(i * L, L)] = jnp.zeros((L,), jnp.float32)
        return carry

    lax.fori_loop(0, NPAD // L, zero, 0)

    ones = jnp.full((L,), 1.0, jnp.float32)

    def body(i, carry):
        idx = dstv[pl.ds(i * L, L)]
        plsc.addupdate_scatter(acc, [idx], ones)
        return carry

    lax.fori_loop(0, EPT1 // L, body, 0)
    pltpu.sync_copy(acc, out_hbm.at[wid])


def _sc_degree(edge_index):
    mesh = plsc.VectorSubcoreMesh(core_axis_name="c", subcore_axis_name="s")
    return pl.kernel(
        _deg_kernel,
        out_type=jax.ShapeDtypeStruct((NC * NS, NPAD), jnp.float32),
        mesh=mesh,
        compiler_params=pltpu.CompilerParams(needs_layout_passes=False),
        scratch_types=[
            pltpu.VMEM((EPT1,), jnp.int32),
            pltpu.VMEM((NPAD,), jnp.float32),
        ],
    )(edge_index)


def _dense_kernel(x_ref, degp_ref, wa, wb, wg, bias, h2_ref, dinv_ref):
    i = pl.program_id(0)
    deg = jnp.sum(degp_ref[...], axis=0) + 1.0
    dinv = lax.rsqrt(deg)
    dinv_ref[0, 0, :] = dinv

    rowids = i * BLK + lax.broadcasted_iota(jnp.int32, (BLK, 1), 0)
    valid = (rowids < N_NODES)[None, :, :]

    x = x_ref[...]
    xa = x[0:T_OUT].reshape(T_OUT * BLK, C)
    xb = x[DILATION:T_IN].reshape(T_OUT * BLK, C)
    p = jnp.dot(xa, wa[...], preferred_element_type=jnp.float32)
    p += jnp.dot(xb, wb[...], preferred_element_type=jnp.float32)
    g = jnp.tanh(p[:, :DC] + bias[0:1, :]) * jax.nn.sigmoid(p[:, DC:] + bias[1:2, :])
    h = jnp.dot(g, wg[...], preferred_element_type=jnp.float32)
    h = h.reshape(T_OUT, BLK, C) * dinv[None, :, None]
    h = jnp.where(valid, h, 0.0)
    for t in range(T_OUT):
        for (q, off, ho, w) in _pieces(t):
            h2_ref[q, :, off:off + w] = h[t, :, ho:ho + w]


def _tc_dense(x, degp, wa, wb, wg, bias):
    full = lambda shape: pl.BlockSpec(shape, lambda i: tuple(0 for _ in shape))
    return pl.pallas_call(
        _dense_kernel,
        grid=(NBLK,),
        in_specs=[
            pl.BlockSpec((T_IN, BLK, C), lambda i: (0, i, 0)),
            pl.BlockSpec((NC * NS, BLK), lambda i: (0, i)),
            full((C, 2 * DC)), full((C, 2 * DC)),
            full((DC, C)), full((2, DC)),
        ],
        out_specs=[
            pl.BlockSpec((Q, BLK, HW), lambda i: (0, i, 0)),
            pl.BlockSpec((1, 1, BLK), lambda i: (i, 0, 0)),
        ],
        out_shape=[
            jax.ShapeDtypeStruct((Q, NPAD, HW), jnp.float32),
            jax.ShapeDtypeStruct((NBLK, 1, BLK), jnp.float32),
        ],
    )(x, degp, wa, wb, wg, bias)


def _agg_kernel(h2_hbm, src_hbm, dst_hbm, agg_hbm, idx_src, idx_dst,
                rows_a, rows_b, sem_a, sem_b, table, acc):
    c = lax.axis_index("c")
    s = lax.axis_index("s")
    srcs = src_hbm.at[s]
    dsts = dst_hbm.at[s]
    rbase = s * RPT

    for p in range(2):
        q = 2 * c + p
        h2q = h2_hbm.at[q]
        pltpu.sync_copy(h2q.at[pl.ds(rbase, RPT), pl.ds(0, QW)],
                        table.at[pl.ds(rbase, RPT)])
        pltpu.sync_copy(h2q.at[pl.ds(rbase, RPT), pl.ds(0, QW)],
                        acc.at[pl.ds(rbase, RPT)])
        plsc.subcore_barrier()

        def outer(b, carry):
            pltpu.sync_copy(srcs.at[pl.ds(b * SUBC, SUBC)], idx_src)
            pltpu.sync_copy(dsts.at[pl.ds(b * SUBC, SUBC)], idx_dst)
            pltpu.async_copy(table.at[idx_src.at[0]], rows_a, sem_a)

            def body(jj, carry2):
                j0 = 2 * jj
                pltpu.async_copy(table.at[idx_src.at[j0 + 1]], rows_b, sem_b)
                pltpu.make_async_copy(table.at[idx_src.at[j0]], rows_a, sem_a).wait()
                pltpu.sync_copy(rows_a, acc.at[idx_dst.at[j0]], add=True)

                @pl.when(j0 + 2 < SUBC)
                def _():
                    pltpu.async_copy(table.at[idx_src.at[j0 + 2]], rows_a, sem_a)

                pltpu.make_async_copy(table.at[idx_src.at[j0 + 1]], rows_b, sem_b).wait()
                pltpu.sync_copy(rows_b, acc.at[idx_dst.at[j0 + 1]], add=True)
                return carry2

            lax.fori_loop(0, SUBC // 2, body, carry)
            return carry

        lax.fori_loop(0, NSUB, outer, 0)
        plsc.subcore_barrier()
        pltpu.sync_copy(acc.at[pl.ds(rbase, RPT)],
                        agg_hbm.at[q].at[pl.ds(rbase, RPT), pl.ds(0, QW)])


def _sc_aggregate(h2, src3d, dst3d):
    mesh = plsc.VectorSubcoreMesh(core_axis_name="c", subcore_axis_name="s")
    return pl.kernel(
        _agg_kernel,
        out_type=jax.ShapeDtypeStruct((Q, NPAD, HW), jnp.float32),
        mesh=mesh,
        compiler_params=pltpu.CompilerParams(use_tc_tiling_on_sc=False),
        scratch_types=[
            pltpu.VMEM((SUBC, K), jnp.int32),
            pltpu.VMEM((SUBC, K), jnp.int32),
            pltpu.VMEM((K, QW), jnp.float32),
            pltpu.VMEM((K, QW), jnp.float32),
            pltpu.SemaphoreType.DMA,
            pltpu.SemaphoreType.DMA,
            pltpu.VMEM_SHARED((NPAD, QW), jnp.float32),
            pltpu.VMEM_SHARED((NPAD, QW), jnp.float32),
        ],
    )(h2, src3d, dst3d)


def _final_kernel(agg_ref, dinv_ref, w2, bias, out_ref):
    dinv_col = dinv_ref[0, 0, :][:, None]
    gb = bias[0:1, :]
    ob = bias[1:2, :]
    for t in range(T_OUT):
        pieces = [agg_ref[q, :, off:off + w] for (q, off, ho, w) in _pieces(t)]
        a = pieces[0] if len(pieces) == 1 else jnp.concatenate(pieces, axis=1)
        a = a * dinv_col + gb
        out_ref[t, :, :] = jnp.dot(a, w2[...], preferred_element_type=jnp.float32) + ob


def _tc_final(agg, dinv2d, w2, bias):
    full = lambda shape: pl.BlockSpec(shape, lambda i: tuple(0 for _ in shape))
    return pl.pallas_call(
        _final_kernel,
        grid=(NBLK,),
        in_specs=[
            pl.BlockSpec((Q, BLK, HW), lambda i: (0, i, 0)),
            pl.BlockSpec((1, 1, BLK), lambda i: (i, 0, 0)),
            full((C, C)), full((2, C)),
        ],
        out_specs=pl.BlockSpec((T_OUT, BLK, C), lambda i: (0, i, 0)),
        out_shape=jax.ShapeDtypeStruct((T_OUT, N_NODES, C), jnp.float32),
    )(agg, dinv2d, w2, bias)


def kernel(input, edge_index, gate1_w, gate1_b, gate2_w, gate2_b, gcn_w, gcn_b, out_w, out_b):
    x = input[0]
    src, dst = _tc_edge_prep(edge_index.reshape(2, 1, E_EDGES))
    src3d = src.reshape(NS, NCHUNK, K)
    dst3d = dst.reshape(NS, NCHUNK, K)

    wa = jnp.concatenate([gate1_w[:, :, 0, 0].T, gate2_w[:, :, 0, 0].T], axis=1)
    wb = jnp.concatenate([gate1_w[:, :, 0, 1].T, gate2_w[:, :, 0, 1].T], axis=1)
    wg = gcn_w.T
    gbias = jnp.stack([gate1_b, gate2_b])
    w2 = out_w[:, :, 0, 0].T
    fbias = jnp.stack([gcn_b, out_b])

    degp = _sc_degree(edge_index.reshape(2, 1, E_EDGES))
    h2, dinv2d = _tc_dense(x, degp, wa, wb, wg, gbias)
    agg = _sc_aggregate(h2, src3d, dst3d)
    out = _tc_final(agg, dinv2d, w2, fbias)
    return out[None]
